# baseline (speedup 1.0000x reference)
import functools

import jax
import jax.numpy as jnp
from jax import lax
from jax.experimental import pallas as pl
from jax.experimental.pallas import tpu as pltpu
from jax.experimental.pallas import tpu_sc as plsc

_CS = 8
_PATCH = 10
_NIDX = 112
_L = 16


def _floor_i32(v):
    c = v.astype(jnp.int32)
    return jnp.where(c.astype(jnp.float32) > v, c - 1, c)


def _build_sc_call(n_view, C, H, W, K):
    nc, ns = 2, 16
    n_workers = nc * ns
    assert K % n_workers == 0
    npt = K // n_workers
    cvec = C // _L
    steps = [8.0 * i / 7.0 - 4.0 for i in range(_CS)]

    def body(feat_hbm, pts_hbm, img_hbm, out_hbm,
             pts_v, img_v, idx_v, patch_v, out_v, gsem):
        wid = lax.axis_index("s") * nc + lax.axis_index("c")
        base_pt = wid * npt
        pltpu.sync_copy(pts_hbm.at[pl.ds(base_pt * 2, npt * 2)],
                        pts_v.at[pl.ds(0, npt * 2)])
        pltpu.sync_copy(img_hbm.at[pl.ds(base_pt, npt)],
                        img_v.at[pl.ds(0, npt)])

        def point_body(p, carry):
            pxy = pts_v[pl.ds(2 * p, _L)]
            pb = img_v[pl.ds(p, _L)]
            x0 = pxy[0]
            y0 = pxy[1]
            b0 = jnp.clip(pb[0], 0, n_view - 1)

            xbase = jnp.clip(_floor_i32(jnp.clip(x0 + steps[0], 0.0, W - 1.0)),
                             0, W - _PATCH)
            ybase = jnp.clip(_floor_i32(jnp.clip(y0 + steps[0], 0.0, H - 1.0)),
                             0, H - _PATCH)
            pixbase = jnp.full((_L,), (b0 * H + ybase) * W + xbase, jnp.int32)

            lane = lax.iota(jnp.int32, _L)
            for s in range(_NIDX // _L):
                e = jnp.where(s * _L + lane < 100, s * _L + lane, 0)
                dy = _floor_i32(e.astype(jnp.float32) * 0.1)
                idx_v[pl.ds(s * _L, _L)] = pixbase + e + dy * (W - _PATCH)

            gather = pltpu.async_copy(feat_hbm.at[idx_v], patch_v, gsem)

            fx, fy, xr, yr = [], [], [], []
            for i in range(_CS):
                xs = jnp.clip(x0 + steps[i], 0.0, W - 1.0)
                ys = jnp.clip(y0 + steps[i], 0.0, H - 1.0)
                xi = _floor_i32(xs)
                yi = _floor_i32(ys)
                fx.append(jnp.full((_L,), xs - xi.astype(jnp.float32)))
                fy.append(jnp.full((_L,), ys - yi.astype(jnp.float32)))
                xr.append(jnp.clip(xi - xbase, 0, _PATCH - 2))
                yr.append(jnp.clip(yi - ybase, 0, _PATCH - 2))

            gather.wait()

            for iy in range(_CS):
                rbase = yr[iy] * _PATCH
                for ix in range(_CS):
                    r = rbase + xr[ix]
                    for j in range(cvec):
                        ds = pl.ds(j * _L, _L)
                        v00 = patch_v[r, ds]
                        v01 = patch_v[r + 1, ds]
                        v10 = patch_v[r + _PATCH, ds]
                        v11 = patch_v[r + _PATCH + 1, ds]
                        t0 = v00 + fy[iy] * (v10 - v00)
                        t1 = v01 + fy[iy] * (v11 - v01)
                        out_v[iy * _CS + ix, ds] = t0 + fx[ix] * (t1 - t0)

            row = (base_pt + p) * (_CS * _CS)
            pltpu.sync_copy(out_v, out_hbm.at[pl.ds(row, _CS * _CS)])
            return carry

        lax.fori_loop(0, npt, point_body, 0)

    mesh = plsc.VectorSubcoreMesh(core_axis_name="c", subcore_axis_name="s")
    return pl.kernel(
        body,
        out_type=jax.ShapeDtypeStruct((K * _CS * _CS, C), jnp.float32),
        mesh=mesh,
        compiler_params=pltpu.CompilerParams(use_tc_tiling_on_sc=False),
        scratch_types=[
            pltpu.VMEM((npt * 2 + _L,), jnp.float32),
            pltpu.VMEM((npt + _L,), jnp.int32),
            pltpu.VMEM((_NIDX,), jnp.int32),
            pltpu.VMEM((_NIDX, C), jnp.float32),
            pltpu.VMEM((_CS * _CS, C), jnp.float32),
            pltpu.SemaphoreType.DMA,
        ],
    )


def kernel(features, sample_points, img_idxs, data):
    B, n_view, C, H, W = features.shape
    _, _, n_track, _ = sample_points.shape
    K = B * n_view * n_track
    table = jnp.transpose(features.reshape(n_view, C, H, W), (0, 2, 3, 1))
    table = table.reshape(n_view * H * W, C)
    pts = sample_points.reshape(-1).astype(jnp.float32)
    img = img_idxs.reshape(-1).astype(jnp.int32)
    out = _build_sc_call(n_view, C, H, W, K)(table, pts, img)
    return out.reshape(B, n_view, n_track, _CS * _CS, C)


# speedup vs baseline: 1.3963x; 1.3963x over previous
import jax
import jax.numpy as jnp
from jax import lax
from jax.experimental import pallas as pl
from jax.experimental.pallas import tpu as pltpu
from jax.experimental.pallas import tpu_sc as plsc

_CS = 8
_PATCH = 10
_L = 16
_PSZ = _PATCH * _PATCH * 96


def _floor_i32(v):
    c = v.astype(jnp.int32)
    return jnp.where(c.astype(jnp.float32) > v, c - 1, c)


def _build_sc_call(n_view, C, H, W, K):
    nc, ns = 2, 16
    n_workers = nc * ns
    assert K % n_workers == 0 and C % _L == 0
    npt = K // n_workers
    cvec = C // _L
    rowf = _PATCH * C
    imrowf = W * C

    def body(feat_hbm, pts_hbm, img_hbm, out_hbm,
             pts_v, img_v, patch0, patch1, out0, out1,
             yl_a, fy_a, gsem0, gsem1, osem0, osem1):
        wid = lax.axis_index("s") * nc + lax.axis_index("c")
        base_pt = wid * npt
        pltpu.sync_copy(pts_hbm.at[pl.ds(base_pt * 2, npt * 2)],
                        pts_v.at[pl.ds(0, npt * 2)])
        pltpu.sync_copy(img_hbm.at[pl.ds(base_pt, npt)],
                        img_v.at[pl.ds(0, npt)])

        def point_base(p):
            pxy = pts_v[pl.ds(2 * p, _L)]
            pb = img_v[pl.ds(p, _L)]
            x0 = pxy[0]
            y0 = pxy[1]
            b0 = jnp.clip(pb[0], 0, n_view - 1)
            xbase = jnp.clip(_floor_i32(jnp.clip(x0 - 4.0, 0.0, W - 1.0)),
                             0, W - _PATCH)
            ybase = jnp.clip(_floor_i32(jnp.clip(y0 - 4.0, 0.0, H - 1.0)),
                             0, H - _PATCH)
            return ((b0 * H + ybase) * W + xbase) * C, x0, y0, xbase, ybase

        def issue_gathers(p, patch, sem):
            start, _, _, _, _ = point_base(p)
            for dy in range(_PATCH):
                pltpu.async_copy(feat_hbm.at[pl.ds(start + dy * imrowf, rowf)],
                                 patch.at[pl.ds(dy * rowf, rowf)], sem)

        lane = lax.iota(jnp.int32, _L)
        lane_f = lane.astype(jnp.float32)

        def compute_point(p, patch, out_v):
            _, x0, y0, xbase, ybase = point_base(p)
            xs = jnp.clip(jnp.full((_L,), x0 - 4.0) + lane_f * (8.0 / 7.0),
                          0.0, W - 1.0)
            ys = jnp.clip(jnp.full((_L,), y0 - 4.0) + lane_f * (8.0 / 7.0),
                          0.0, H - 1.0)
            xi = _floor_i32(xs)
            yi = _floor_i32(ys)
            fxv = xs - xi.astype(jnp.float32)
            fyv = ys - yi.astype(jnp.float32)
            xlv = jnp.clip(xi - jnp.full((_L,), xbase), 0, _PATCH - 2)
            ylv = jnp.clip(yi - jnp.full((_L,), ybase), 0, _PATCH - 2)
            yl_a[pl.ds(0, _L)] = ylv
            fy_a[pl.ds(0, _L)] = fyv
            xoffs = [xlv[i] * C for i in range(_CS)]
            fxs = [jnp.full((_L,), fxv[i]) for i in range(_CS)]

            def row_body(iy, carry):
                rbase = yl_a[pl.ds(iy, _L)][0] * rowf
                fy = jnp.full((_L,), fy_a[pl.ds(iy, _L)][0])
                for ix in range(_CS):
                    off = rbase + xoffs[ix]
                    for j in range(cvec):
                        v00 = patch[pl.ds(off + j * _L, _L)]
                        v01 = patch[pl.ds(off + C + j * _L, _L)]
                        v10 = patch[pl.ds(off + rowf + j * _L, _L)]
                        v11 = patch[pl.ds(off + rowf + C + j * _L, _L)]
                        t0 = v00 + fy * (v10 - v00)
                        t1 = v01 + fy * (v11 - v01)
                        out_v[iy * _CS + ix, pl.ds(j * _L, _L)] = (
                            t0 + fxs[ix] * (t1 - t0))
                return carry

            lax.fori_loop(0, _CS, row_body, 0)

        bufs = ((patch0, out0, gsem0, osem0), (patch1, out1, gsem1, osem1))

        issue_gathers(0, patch0, gsem0)

        def pair_body(p2, carry):
            for b in range(2):
                patch, out_v, gsem, osem = bufs[b]
                n_patch, _, n_gsem, _ = bufs[1 - b]
                pcur = 2 * p2 + b
                pnext = jnp.minimum(pcur + 1, npt - 1)
                issue_gathers(pnext, n_patch, n_gsem)
                pltpu.make_async_copy(feat_hbm.at[pl.ds(0, _PSZ)],
                                      patch, gsem).wait()

                @pl.when(pcur >= 2)
                def _():
                    pltpu.make_async_copy(out_hbm.at[pl.ds(0, _CS * _CS)],
                                          out_v, osem).wait()

                compute_point(pcur, patch, out_v)
                row = (base_pt + pcur) * (_CS * _CS)
                pltpu.async_copy(out_v, out_hbm.at[pl.ds(row, _CS * _CS)],
                                 osem)
            return carry

        lax.fori_loop(0, npt // 2, pair_body, 0)

        pltpu.make_async_copy(out_hbm.at[pl.ds(0, _CS * _CS)], out0,
                              osem0).wait()
        pltpu.make_async_copy(out_hbm.at[pl.ds(0, _CS * _CS)], out1,
                              osem1).wait()
        pltpu.make_async_copy(feat_hbm.at[pl.ds(0, _PSZ)], patch0,
                              gsem0).wait()

    mesh = plsc.VectorSubcoreMesh(core_axis_name="c", subcore_axis_name="s")
    return pl.kernel(
        body,
        out_type=jax.ShapeDtypeStruct((K * _CS * _CS, C), jnp.float32),
        mesh=mesh,
        compiler_params=pltpu.CompilerParams(use_tc_tiling_on_sc=False),
        scratch_types=[
            pltpu.VMEM((npt * 2 + _L,), jnp.float32),
            pltpu.VMEM((npt + _L,), jnp.int32),
            pltpu.VMEM((_PSZ,), jnp.float32),
            pltpu.VMEM((_PSZ,), jnp.float32),
            pltpu.VMEM((_CS * _CS, C), jnp.float32),
            pltpu.VMEM((_CS * _CS, C), jnp.float32),
            pltpu.VMEM((2 * _L,), jnp.int32),
            pltpu.VMEM((2 * _L,), jnp.float32),
            pltpu.SemaphoreType.DMA,
            pltpu.SemaphoreType.DMA,
            pltpu.SemaphoreType.DMA,
            pltpu.SemaphoreType.DMA,
        ],
    )


def kernel(features, sample_points, img_idxs, data):
    B, n_view, C, H, W = features.shape
    _, _, n_track, _ = sample_points.shape
    K = B * n_view * n_track
    table = jnp.transpose(features.reshape(n_view, C, H, W), (0, 2, 3, 1))
    table = table.reshape(n_view * H * W * C)
    pts = sample_points.reshape(-1).astype(jnp.float32)
    img = img_idxs.reshape(-1).astype(jnp.int32)
    out = _build_sc_call(n_view, C, H, W, K)(table, pts, img)
    return out.reshape(B, n_view, n_track, _CS * _CS, C)


# speedup vs baseline: 1.3988x; 1.0018x over previous
import jax
import jax.numpy as jnp
from jax import lax
from jax.experimental import pallas as pl
from jax.experimental.pallas import tpu as pltpu
from jax.experimental.pallas import tpu_sc as plsc

_CS = 8
_PATCH = 10
_L = 16


def _floor_i32(v):
    c = v.astype(jnp.int32)
    return jnp.where(c.astype(jnp.float32) > v, c - 1, c)


def _tp_body(x_ref, o_ref):
    o_ref[...] = jnp.transpose(x_ref[...], (0, 2, 1))


def _build_tp_call(n_view, C, HW, blk):
    return pl.pallas_call(
        _tp_body,
        grid=(n_view, HW // blk),
        in_specs=[pl.BlockSpec((1, C, blk), lambda v, j: (v, 0, j))],
        out_specs=pl.BlockSpec((1, blk, C), lambda v, j: (v, j, 0)),
        out_shape=jax.ShapeDtypeStruct((n_view, HW, C), jnp.float32),
    )


def _build_sc_call(n_view, C, H, W, K):
    nc, ns = 2, 16
    n_workers = nc * ns
    assert K % n_workers == 0 and C % _L == 0
    npt = K // n_workers
    cvec = C // _L

    def body(feat_hbm, pts_hbm, img_hbm, out_hbm,
             pts_v, img_v, patch0, patch1, out0, out1,
             yl_a, fy_a, gsem0, gsem1, osem0, osem1):
        wid = lax.axis_index("s") * nc + lax.axis_index("c")
        base_pt = wid * npt
        pltpu.sync_copy(pts_hbm.at[pl.ds(base_pt * 2, npt * 2)],
                        pts_v.at[pl.ds(0, npt * 2)])
        pltpu.sync_copy(img_hbm.at[pl.ds(base_pt, npt)],
                        img_v.at[pl.ds(0, npt)])

        def point_base(p):
            pxy = pts_v[pl.ds(2 * p, _L)]
            pb = img_v[pl.ds(p, _L)]
            x0 = pxy[0]
            y0 = pxy[1]
            b0 = jnp.clip(pb[0], 0, n_view - 1)
            xbase = jnp.clip(_floor_i32(jnp.clip(x0 - 4.0, 0.0, W - 1.0)),
                             0, W - _PATCH)
            ybase = jnp.clip(_floor_i32(jnp.clip(y0 - 4.0, 0.0, H - 1.0)),
                             0, H - _PATCH)
            return b0 * H + ybase, xbase * C, x0, y0, xbase, ybase

        def issue_gather(p, patch, sem):
            r0, c0, _, _, _, _ = point_base(p)
            pltpu.async_copy(
                feat_hbm.at[pl.ds(r0, _PATCH), pl.ds(c0, _PATCH * C)],
                patch, sem)

        lane = lax.iota(jnp.int32, _L)
        lane_f = lane.astype(jnp.float32)

        def compute_point(p, patch, out_v):
            _, _, x0, y0, xbase, ybase = point_base(p)
            xs = jnp.clip(jnp.full((_L,), x0 - 4.0) + lane_f * (8.0 / 7.0),
                          0.0, W - 1.0)
            ys = jnp.clip(jnp.full((_L,), y0 - 4.0) + lane_f * (8.0 / 7.0),
                          0.0, H - 1.0)
            xi = _floor_i32(xs)
            yi = _floor_i32(ys)
            fxv = xs - xi.astype(jnp.float32)
            fyv = ys - yi.astype(jnp.float32)
            xlv = jnp.clip(xi - jnp.full((_L,), xbase), 0, _PATCH - 2)
            ylv = jnp.clip(yi - jnp.full((_L,), ybase), 0, _PATCH - 2)
            yl_a[pl.ds(0, _L)] = ylv
            fy_a[pl.ds(0, _L)] = fyv
            xoffs = [xlv[i] * C for i in range(_CS)]
            fxs = [jnp.full((_L,), fxv[i]) for i in range(_CS)]

            def row_body(iy, carry):
                r = yl_a[pl.ds(iy, _L)][0]
                fy = jnp.full((_L,), fy_a[pl.ds(iy, _L)][0])
                for ix in range(_CS):
                    off = xoffs[ix]
                    for j in range(cvec):
                        v00 = patch[r, pl.ds(off + j * _L, _L)]
                        v01 = patch[r, pl.ds(off + C + j * _L, _L)]
                        v10 = patch[r + 1, pl.ds(off + j * _L, _L)]
                        v11 = patch[r + 1, pl.ds(off + C + j * _L, _L)]
                        t0 = v00 + fy * (v10 - v00)
                        t1 = v01 + fy * (v11 - v01)
                        out_v[iy * _CS + ix, pl.ds(j * _L, _L)] = (
                            t0 + fxs[ix] * (t1 - t0))
                return carry

            lax.fori_loop(0, _CS, row_body, 0)

        bufs = ((patch0, out0, gsem0, osem0), (patch1, out1, gsem1, osem1))

        issue_gather(0, patch0, gsem0)

        def pair_body(p2, carry):
            for b in range(2):
                patch, out_v, gsem, osem = bufs[b]
                n_patch, _, n_gsem, _ = bufs[1 - b]
                pcur = 2 * p2 + b
                pnext = jnp.minimum(pcur + 1, npt - 1)
                issue_gather(pnext, n_patch, n_gsem)
                pltpu.make_async_copy(
                    feat_hbm.at[pl.ds(0, _PATCH), pl.ds(0, _PATCH * C)],
                    patch, gsem).wait()

                @pl.when(pcur >= 2)
                def _():
                    pltpu.make_async_copy(out_hbm.at[pl.ds(0, _CS * _CS)],
                                          out_v, osem).wait()

                compute_point(pcur, patch, out_v)
                row = (base_pt + pcur) * (_CS * _CS)
                pltpu.async_copy(out_v, out_hbm.at[pl.ds(row, _CS * _CS)],
                                 osem)
            return carry

        lax.fori_loop(0, npt // 2, pair_body, 0)

        pltpu.make_async_copy(out_hbm.at[pl.ds(0, _CS * _CS)], out0,
                              osem0).wait()
        pltpu.make_async_copy(out_hbm.at[pl.ds(0, _CS * _CS)], out1,
                              osem1).wait()
        pltpu.make_async_copy(
            feat_hbm.at[pl.ds(0, _PATCH), pl.ds(0, _PATCH * C)],
            patch0, gsem0).wait()

    mesh = plsc.VectorSubcoreMesh(core_axis_name="c", subcore_axis_name="s")
    return pl.kernel(
        body,
        out_type=jax.ShapeDtypeStruct((K * _CS * _CS, C), jnp.float32),
        mesh=mesh,
        compiler_params=pltpu.CompilerParams(use_tc_tiling_on_sc=False),
        scratch_types=[
            pltpu.VMEM((npt * 2 + _L,), jnp.float32),
            pltpu.VMEM((npt + _L,), jnp.int32),
            pltpu.VMEM((_PATCH, _PATCH * 96), jnp.float32),
            pltpu.VMEM((_PATCH, _PATCH * 96), jnp.float32),
            pltpu.VMEM((_CS * _CS, 96), jnp.float32),
            pltpu.VMEM((_CS * _CS, 96), jnp.float32),
            pltpu.VMEM((2 * _L,), jnp.int32),
            pltpu.VMEM((2 * _L,), jnp.float32),
            pltpu.SemaphoreType.DMA,
            pltpu.SemaphoreType.DMA,
            pltpu.SemaphoreType.DMA,
            pltpu.SemaphoreType.DMA,
        ],
    )


def kernel(features, sample_points, img_idxs, data):
    B, n_view, C, H, W = features.shape
    _, _, n_track, _ = sample_points.shape
    K = B * n_view * n_track
    table = _build_tp_call(n_view, C, H * W, 512)(
        features.reshape(n_view, C, H * W))
    table = table.reshape(n_view * H, W * C)
    pts = sample_points.reshape(-1).astype(jnp.float32)
    img = img_idxs.reshape(-1).astype(jnp.int32)
    out = _build_sc_call(n_view, C, H, W, K)(table, pts, img)
    return out.reshape(B, n_view, n_track, _CS * _CS, C)
